# baseline (speedup 1.0000x reference)
import functools

import numpy as np
import jax
import jax.numpy as jnp
from jax import lax
from jax.experimental import pallas as pl
from jax.experimental.pallas import tpu as pltpu
from jax.experimental.pallas import tpu_sc as plsc

_SCALING = float(np.power(2.0, np.random.default_rng(seed=42).uniform(-1, 1)))
_L = 220500
_OUTPUT_SIZE = int(_L * _SCALING)
_OFFSET = (_OUTPUT_SIZE - _L) // 2
_SCALING_F32 = np.float32(_SCALING)

_NC, _NS = 2, 16
_NW = _NC * _NS
_ROWS = 16
_C = 6912
_PAD = _C * _NW
_ITERS = _C // 16
_W0 = 34832
_WSTRIDE = 4728
_WIN = 4752


def _interp_body(wav_hbm, out_hbm, win_v, out_v):
    wid = lax.axis_index("s") * _NC + lax.axis_index("c")
    start = _W0 + wid * _WSTRIDE
    jbase0 = wid * _C
    lane = lax.iota(jnp.int32, 16)

    for row in range(_ROWS):
        b, ch = divmod(row, 2)
        pltpu.sync_copy(
            wav_hbm.at[1, b, ch, pl.ds(start, _WIN)],
            win_v.at[pl.ds(row * _WIN, _WIN)],
        )

    for row in range(_ROWS):
        b, ch = divmod(row, 2)
        rowbase = row * _WIN

        def body(it, jbase, rowbase=rowbase):
            jv = jbase + lane
            src = (jv + _OFFSET).astype(jnp.float32) / _SCALING_F32
            i0 = src.astype(jnp.int32)
            r = src - i0.astype(jnp.float32)
            li0 = i0 - start + rowbase
            x0 = plsc.load_gather(win_v, [li0])
            x1 = plsc.load_gather(win_v, [li0 + 1])
            out_v[pl.ds(it * 16, 16)] = x0 + r * (x1 - x0)
            return jbase + 16

        lax.fori_loop(0, _ITERS, body, jbase0, unroll=4)
        pltpu.sync_copy(out_v, out_hbm.at[b, ch, pl.ds(jbase0, _C)])


@jax.jit
def _sc_interp(wav):
    mesh = plsc.VectorSubcoreMesh(core_axis_name="c", subcore_axis_name="s")
    f = functools.partial(
        pl.kernel,
        mesh=mesh,
        out_type=jax.ShapeDtypeStruct((8, 2, _PAD), jnp.float32),
        scratch_types=[
            pltpu.VMEM((_ROWS * _WIN,), jnp.float32),
            pltpu.VMEM((_C,), jnp.float32),
        ],
        compiler_params=pltpu.CompilerParams(
            use_tc_tiling_on_sc=False, needs_layout_passes=False
        ),
    )(_interp_body)
    return f(wav)


_CB = 8192
_NCB = (_L + _CB - 1) // _CB


def _assemble_body(wav_ref, scaled_ref, out_ref):
    out_ref[0] = wav_ref[0]
    out_ref[1] = scaled_ref[...]
    out_ref[2] = wav_ref[2]
    out_ref[3] = wav_ref[3]


@jax.jit
def _assemble(wav, scaled):
    return pl.pallas_call(
        _assemble_body,
        out_shape=jax.ShapeDtypeStruct((4, 8, 2, _L), jnp.float32),
        grid=(8, _NCB),
        in_specs=[
            pl.BlockSpec((4, 1, 2, _CB), lambda b, k: (0, b, 0, k)),
            pl.BlockSpec((1, 2, _CB), lambda b, k: (b, 0, k)),
        ],
        out_specs=pl.BlockSpec((4, 1, 2, _CB), lambda b, k: (0, b, 0, k)),
    )(wav, scaled)


def kernel(wav):
    return _assemble(wav, _sc_interp(wav))


# speedup vs baseline: 4.0153x; 4.0153x over previous
import functools

import numpy as np
import jax
import jax.numpy as jnp
from jax import lax
from jax.experimental import pallas as pl
from jax.experimental.pallas import tpu as pltpu
from jax.experimental.pallas import tpu_sc as plsc

_SCALING = float(np.power(2.0, np.random.default_rng(seed=42).uniform(-1, 1)))
_L = 220500
_OUTPUT_SIZE = int(_L * _SCALING)
_OFFSET = (_OUTPUT_SIZE - _L) // 2
_SCALING_F32 = np.float32(_SCALING)

_NC, _NS = 2, 16
_NW = _NC * _NS
_ROWS = 16
_C = 6912
_PAD = _C * _NW
_ITERS = _C // 16
_W0 = 34560
_WSTRIDE = 4736
_WIN = 5120


def _interp_body(wav_hbm, out_hbm, win_v, out_v):
    wid = lax.axis_index("s") * _NC + lax.axis_index("c")
    start = _W0 + wid * _WSTRIDE
    jbase0 = wid * _C
    lane = lax.iota(jnp.int32, 16)

    for b in range(8):
        pltpu.sync_copy(wav_hbm.at[1, b, :, pl.ds(start, _WIN)], win_v)

        for ch in range(2):
            chs = jnp.full((16,), ch, jnp.int32)

            def body(it, jbase, ch=ch, chs=chs):
                jv = jbase + lane
                src = (jv + _OFFSET).astype(jnp.float32) / _SCALING_F32
                i0 = src.astype(jnp.int32)
                r = src - i0.astype(jnp.float32)
                li0 = i0 - start
                x0 = plsc.load_gather(win_v, [chs, li0])
                x1 = plsc.load_gather(win_v, [chs, li0 + 1])
                out_v[ch, pl.ds(it * 16, 16)] = x0 + r * (x1 - x0)
                return jbase + 16

            lax.fori_loop(0, _ITERS, body, jbase0, unroll=4)

        pltpu.sync_copy(out_v, out_hbm.at[b, :, pl.ds(jbase0, _C)])


@jax.jit
def _sc_interp(wav):
    mesh = plsc.VectorSubcoreMesh(core_axis_name="c", subcore_axis_name="s")
    f = functools.partial(
        pl.kernel,
        mesh=mesh,
        out_type=jax.ShapeDtypeStruct((8, 2, _PAD), jnp.float32),
        scratch_types=[
            pltpu.VMEM((2, _WIN), jnp.float32),
            pltpu.VMEM((2, _C), jnp.float32),
        ],
        compiler_params=pltpu.CompilerParams(needs_layout_passes=False),
    )(_interp_body)
    return f(wav)


_CB = 8192
_NCB = (_L + _CB - 1) // _CB


def _assemble_body(wav_ref, scaled_ref, out_ref):
    out_ref[0] = wav_ref[0]
    out_ref[1] = scaled_ref[...]
    out_ref[2] = wav_ref[2]
    out_ref[3] = wav_ref[3]


@jax.jit
def _assemble(wav, scaled):
    return pl.pallas_call(
        _assemble_body,
        out_shape=jax.ShapeDtypeStruct((4, 8, 2, _L), jnp.float32),
        grid=(8, _NCB),
        in_specs=[
            pl.BlockSpec((4, 1, 2, _CB), lambda b, k: (0, b, 0, k)),
            pl.BlockSpec((1, 2, _CB), lambda b, k: (b, 0, k)),
        ],
        out_specs=pl.BlockSpec((4, 1, 2, _CB), lambda b, k: (0, b, 0, k)),
    )(wav, scaled)


def kernel(wav):
    return _assemble(wav, _sc_interp(wav))
